# Initial kernel scaffold; baseline (speedup 1.0000x reference)
import jax
import jax.numpy as jnp
from jax.experimental import pallas as pl


def kernel(first_index, second_index, edge_index, emb_user, emb_item, W_gcn, b_gcn, W1, b1, W2, b2):
    raise NotImplementedError("write your pallas kernel here")


# speedup vs baseline: 3.2277x; 3.2277x over previous
import functools

import jax
import jax.numpy as jnp
from jax import lax
from jax.experimental import pallas as pl
from jax.experimental.pallas import tpu as pltpu
from jax.experimental.pallas import tpu_sc as plsc

NC = 2
NS = 16
LN = 16
CH = 128
EG = 16


def _sc_mesh():
    return plsc.VectorSubcoreMesh(
        core_axis_name="c", subcore_axis_name="s", num_cores=NC, num_subcores=NS
    )


def _fill_rows(idxb, base, lanes):
    def fi(i, _):
        idxb[pl.ds(i * LN, LN)] = base + i * LN + lanes
        return 0

    lax.fori_loop(0, CH // LN, fi, 0)


def _zero_slice(zbuf, shared, gidx, s, rpt, lanes):
    def zc(t, _):
        _fill_rows(gidx, s * rpt + t * CH, lanes)
        pltpu.sync_copy(zbuf, shared.at[gidx])
        return 0

    lax.fori_loop(0, rpt // CH, zc, 0)


def _localize(buf2d, rows, lo, npc, lanes_unused):
    vec = CH // LN

    def txb(i, _):
        j = i // vec
        k = (i % vec) * LN
        v = buf2d[j, pl.ds(k, LN)] - lo
        ok = (v >= 0) & (v < npc)
        buf2d[j, pl.ds(k, LN)] = jnp.where(ok, v, npc)
        return 0

    lax.fori_loop(0, rows * vec, txb, 0)


def _make_deg_kernel(npad, nept):
    nec = nept // CH
    ngr = nec // EG
    npc = npad // NC
    rpt = npc // NS

    @functools.partial(
        pl.kernel,
        out_type=jax.ShapeDtypeStruct((npad, CH), jnp.float32),
        mesh=_sc_mesh(),
        scratch_types=[
            pltpu.VMEM((EG, CH), jnp.int32),
            pltpu.VMEM((CH, CH), jnp.float32),
            pltpu.VMEM((CH, CH), jnp.float32),
            pltpu.VMEM((CH,), jnp.int32),
            pltpu.VMEM_SHARED((npad // NC + 8, CH), jnp.float32),
        ],
    )
    def deg_kernel(edset, out_hbm, sgrp, ones_b, zbuf, gidx, histS):
        c = lax.axis_index("c")
        s = lax.axis_index("s")
        lo = c * npc
        lanes = lax.broadcasted_iota(jnp.int32, (LN,), 0)
        zeros = jnp.zeros((LN,), jnp.float32)
        npl = CH // LN

        def fb(i, _):
            zbuf[i // npl, pl.ds((i % npl) * LN, LN)] = zeros
            ones_b[i // npl, pl.ds((i % npl) * LN, LN)] = zeros + 1.0
            return 0

        lax.fori_loop(0, CH * npl, fb, 0)
        _zero_slice(zbuf, histS, gidx, s, rpt, lanes)

        @pl.when(s == 0)
        def _():
            _fill_rows(gidx, npc - CH + 8, lanes)
            pltpu.sync_copy(zbuf, histS.at[gidx])

        plsc.subcore_barrier()

        def egrp(g, _):
            pltpu.sync_copy(edset.at[pl.ds((s * ngr + g) * EG, EG)], sgrp)
            _localize(sgrp, EG, lo, npc, lanes)

            def body(j, _):
                pltpu.sync_copy(ones_b, histS.at[sgrp.at[j]], add=True)
                return 0

            lax.fori_loop(0, EG, body, 0)
            return 0

        lax.fori_loop(0, ngr, egrp, 0)
        plsc.subcore_barrier()

        def co(t, _):
            r = s * rpt + t * CH
            _fill_rows(gidx, r, lanes)
            pltpu.sync_copy(histS.at[gidx], zbuf)
            pltpu.sync_copy(zbuf, out_hbm.at[pl.ds(c * npc + r, CH)])
            return 0

        lax.fori_loop(0, rpt // CH, co, 0)

    return deg_kernel


def _make_norm_scale(npad, d, rb):
    def body(hist_ref, x_ref, y_ref, n128_ref):
        deg = hist_ref[...][:, 0]
        nrm = lax.rsqrt(jnp.maximum(deg, 1.0))
        n128_ref[...] = jnp.broadcast_to(nrm[:, None], (rb, CH))
        y_ref[...] = x_ref[...] * nrm[:, None]

    return pl.pallas_call(
        body,
        grid=(npad // rb,),
        in_specs=[
            pl.BlockSpec((rb, CH), lambda i: (i, 0)),
            pl.BlockSpec((rb, d), lambda i: (i, 0)),
        ],
        out_specs=[
            pl.BlockSpec((rb, d), lambda i: (i, 0)),
            pl.BlockSpec((rb, CH), lambda i: (i, 0)),
        ],
        out_shape=[
            jax.ShapeDtypeStruct((npad, d), jnp.float32),
            jax.ShapeDtypeStruct((npad, CH), jnp.float32),
        ],
    )


def _make_agg_kernel(npad, d, ems, bt):
    nec = ems // CH
    ngr = nec // EG
    ngt = bt // CH // NS
    npc = npad // NC
    rpt = npc // NS

    @functools.partial(
        pl.kernel,
        out_type=jax.ShapeDtypeStruct((NC, bt, d), jnp.float32),
        mesh=_sc_mesh(),
        scratch_types=[
            pltpu.VMEM((EG, CH), jnp.int32),
            pltpu.VMEM((EG, CH), jnp.int32),
            pltpu.VMEM((CH, d), jnp.float32),
            pltpu.VMEM((CH, d), jnp.float32),
            pltpu.VMEM((CH,), jnp.int32),
            pltpu.VMEM_SHARED((npad // NC + 8, d), jnp.float32),
            pltpu.SemaphoreType.DMA,
            pltpu.SemaphoreType.DMA,
        ],
    )
    def agg_kernel(y, esrc, edst, idxg, aggg_out,
                   sgrp, dgrp, buf, zbuf, gidx, aggS, sem0, sem1):
        c = lax.axis_index("c")
        s = lax.axis_index("s")
        lo = c * npc
        lanes = lax.broadcasted_iota(jnp.int32, (LN,), 0)
        zeros = jnp.zeros((LN,), jnp.float32)
        npl = d // LN

        def zb(i, _):
            zbuf[i // npl, pl.ds((i % npl) * LN, LN)] = zeros
            return 0

        lax.fori_loop(0, CH * npl, zb, 0)
        _zero_slice(zbuf, aggS, gidx, s, rpt, lanes)

        @pl.when(s == 0)
        def _():
            _fill_rows(gidx, npc - CH + 8, lanes)
            pltpu.sync_copy(zbuf, aggS.at[gidx])

        plsc.subcore_barrier()

        def egrp(g, _):
            pltpu.sync_copy(esrc.at[s, pl.ds(g * EG, EG)], sgrp)
            pltpu.sync_copy(edst.at[s, pl.ds(g * EG, EG)], dgrp)
            _localize(dgrp, EG, lo, npc, lanes)

            def ebody(j, _):
                pltpu.async_copy(y.at[sgrp.at[j]], buf, sem0).wait()
                pltpu.sync_copy(buf, aggS.at[dgrp.at[j]], add=True)
                return 0

            lax.fori_loop(0, EG, ebody, 0)
            return 0

        lax.fori_loop(0, ngr, egrp, 0)
        plsc.subcore_barrier()

        def gbody(k, _):
            j = s * ngt + k
            pltpu.sync_copy(idxg.at[j], gidx)

            def lxb(i, _):
                v = gidx[pl.ds(i * LN, LN)] - lo
                ok = (v >= 0) & (v < npc)
                gidx[pl.ds(i * LN, LN)] = jnp.where(ok, v, npc)
                return 0

            lax.fori_loop(0, CH // LN, lxb, 0)
            pltpu.async_copy(aggS.at[gidx], zbuf, sem1).wait()
            pltpu.sync_copy(zbuf, aggg_out.at[c, pl.ds(j * CH, CH)])
            return 0

        lax.fori_loop(0, ngt, gbody, 0)

    return agg_kernel


def _make_normg_kernel(npad, bt):
    ngt = bt // CH // NS
    half = ngt // NC

    @functools.partial(
        pl.kernel,
        out_type=jax.ShapeDtypeStruct((bt, CH), jnp.float32),
        mesh=_sc_mesh(),
        scratch_types=[
            pltpu.VMEM((CH,), jnp.int32),
            pltpu.VMEM((CH, CH), jnp.float32),
            pltpu.SemaphoreType.DMA,
        ],
    )
    def normg_kernel(n128, idxg, out_hbm, gidx, nbuf, sem):
        c = lax.axis_index("c")
        s = lax.axis_index("s")

        def gbody(k, _):
            j = s * ngt + c * half + k
            pltpu.sync_copy(idxg.at[j], gidx)
            pltpu.async_copy(n128.at[gidx], nbuf, sem).wait()
            pltpu.sync_copy(nbuf, out_hbm.at[pl.ds(j * CH, CH)])
            return 0

        lax.fori_loop(0, half, gbody, 0)

    return normg_kernel


def _make_mlp(bt, d, h1, npc):
    b = bt // 2

    def body(aggg_ref, normg_ref, idxg_ref, wg_ref, bg_ref, w1_ref, b1_ref,
             w2_ref, b2_ref, out_ref):
        nrm = normg_ref[...][:, :1]
        own0 = idxg_ref[...] < npc
        a = jnp.where(own0, aggg_ref[0], aggg_ref[1]) * nrm
        h = (
            jnp.dot(a, wg_ref[...], preferred_element_type=jnp.float32)
            + bg_ref[...]
        )
        h = jnp.maximum(h, 0.0)
        hf = h[:b]
        hs = h[b:]
        z = (
            jnp.dot(hf, w1_ref[:d, :], preferred_element_type=jnp.float32)
            + jnp.dot(hs, w1_ref[d:, :], preferred_element_type=jnp.float32)
            + b1_ref[...]
        )
        z = jnp.maximum(z, 0.0)
        out_ref[...] = (
            jnp.dot(z, w2_ref[...], preferred_element_type=jnp.float32)
            + b2_ref[...]
        )

    return pl.pallas_call(
        body,
        out_shape=jax.ShapeDtypeStruct((b, 1), jnp.float32),
    )


def kernel(first_index, second_index, edge_index, emb_user, emb_item,
           W_gcn, b_gcn, W1, b1, W2, b2):
    nu, d = emb_user.shape
    ni = emb_item.shape[0]
    n = nu + ni
    e = edge_index.shape[1]
    bsz = first_index.shape[0]
    h1 = W1.shape[1]

    npad = -(-(n + 1) // (NC * NS * CH)) * (NC * NS * CH)
    epad = -(-e // (NS * CH * EG // 2)) * (NS * CH * EG // 2)
    ems = 2 * epad // NS
    nept = 2 * epad // NS
    bt = 2 * bsz

    idt = jnp.int32
    pad_e = jnp.full((epad - e,), n, dtype=idt)
    e0 = jnp.concatenate([edge_index[0].astype(idt), pad_e])
    e1 = jnp.concatenate([edge_index[1].astype(idt), pad_e])
    edset = jnp.concatenate([e0, e1]).reshape(NS * nept // CH, CH)
    esrc = jnp.concatenate(
        [e0.reshape(NS, -1), e1.reshape(NS, -1)], axis=1
    ).reshape(NS, ems // CH, CH)
    edst = jnp.concatenate(
        [e1.reshape(NS, -1), e0.reshape(NS, -1)], axis=1
    ).reshape(NS, ems // CH, CH)
    x = jnp.concatenate(
        [emb_user, emb_item, jnp.zeros((npad - n, d), jnp.float32)]
    )
    idxg_flat = jnp.concatenate(
        [first_index.astype(idt), second_index.astype(idt) + nu]
    )

    hist = _make_deg_kernel(npad, nept)(edset)
    y, n128 = _make_norm_scale(npad, d, 2048)(hist, x)
    aggg = _make_agg_kernel(npad, d, ems, bt)(
        y, esrc, edst, idxg_flat.reshape(bt // CH, CH)
    )
    normg = _make_normg_kernel(npad, bt)(n128, idxg_flat.reshape(bt // CH, CH))
    out = _make_mlp(bt, d, h1, npad // NC)(
        aggg,
        normg,
        idxg_flat.reshape(bt, 1),
        W_gcn,
        b_gcn.reshape(1, d),
        W1,
        b1.reshape(1, h1),
        W2,
        b2.reshape(1, 1),
    )
    return out.reshape(bsz)


# speedup vs baseline: 3.4111x; 1.0568x over previous
import functools

import jax
import jax.numpy as jnp
from jax import lax
from jax.experimental import pallas as pl
from jax.experimental.pallas import tpu as pltpu
from jax.experimental.pallas import tpu_sc as plsc

NC = 2
NS = 16
LN = 16
CH = 128
EG = 16


def _sc_mesh():
    return plsc.VectorSubcoreMesh(
        core_axis_name="c", subcore_axis_name="s", num_cores=NC, num_subcores=NS
    )


def _fill_rows(idxb, base, lanes):
    def fi(i, _):
        idxb[pl.ds(i * LN, LN)] = base + i * LN + lanes
        return 0

    lax.fori_loop(0, CH // LN, fi, 0)


def _zero_slice(zbuf, shared, gidx, s, rpt, lanes):
    def zc(t, _):
        _fill_rows(gidx, s * rpt + t * CH, lanes)
        pltpu.sync_copy(zbuf, shared.at[gidx])
        return 0

    lax.fori_loop(0, rpt // CH, zc, 0)


def _localize(buf2d, rows, lo, npc, lanes_unused):
    vec = CH // LN

    def txb(i, _):
        j = i // vec
        k = (i % vec) * LN
        v = buf2d[j, pl.ds(k, LN)] - lo
        ok = (v >= 0) & (v < npc)
        buf2d[j, pl.ds(k, LN)] = jnp.where(ok, v, npc)
        return 0

    lax.fori_loop(0, rows * vec, txb, 0)


def _make_deg_kernel(npad, nept):
    nec = nept // CH
    ngr = nec // EG
    npc = npad // NC
    rpt = npc // NS

    @functools.partial(
        pl.kernel,
        out_type=jax.ShapeDtypeStruct((npad, CH), jnp.float32),
        mesh=_sc_mesh(),
        scratch_types=[
            pltpu.VMEM((EG, CH), jnp.int32),
            pltpu.VMEM((CH, CH), jnp.float32),
            pltpu.VMEM((CH, CH), jnp.float32),
            pltpu.VMEM((CH,), jnp.int32),
            pltpu.VMEM_SHARED((npad // NC + 8, CH), jnp.float32),
        ],
    )
    def deg_kernel(edset, out_hbm, sgrp, ones_b, zbuf, gidx, histS):
        c = lax.axis_index("c")
        s = lax.axis_index("s")
        lo = c * npc
        lanes = lax.broadcasted_iota(jnp.int32, (LN,), 0)
        zeros = jnp.zeros((LN,), jnp.float32)
        npl = CH // LN

        def fb(i, _):
            zbuf[i // npl, pl.ds((i % npl) * LN, LN)] = zeros
            ones_b[i // npl, pl.ds((i % npl) * LN, LN)] = zeros + 1.0
            return 0

        lax.fori_loop(0, CH * npl, fb, 0)
        _zero_slice(zbuf, histS, gidx, s, rpt, lanes)

        @pl.when(s == 0)
        def _():
            _fill_rows(gidx, npc - CH + 8, lanes)
            pltpu.sync_copy(zbuf, histS.at[gidx])

        plsc.subcore_barrier()

        def egrp(g, _):
            pltpu.sync_copy(edset.at[pl.ds((s * ngr + g) * EG, EG)], sgrp)
            _localize(sgrp, EG, lo, npc, lanes)

            def body(j, _):
                pltpu.sync_copy(ones_b, histS.at[sgrp.at[j]], add=True)
                return 0

            lax.fori_loop(0, EG, body, 0)
            return 0

        lax.fori_loop(0, ngr, egrp, 0)
        plsc.subcore_barrier()

        def co(t, _):
            r = s * rpt + t * CH
            _fill_rows(gidx, r, lanes)
            pltpu.sync_copy(histS.at[gidx], zbuf)
            pltpu.sync_copy(zbuf, out_hbm.at[pl.ds(c * npc + r, CH)])
            return 0

        lax.fori_loop(0, rpt // CH, co, 0)

    return deg_kernel


def _make_norm_scale(npad, d, rb):
    def body(hist_ref, x_ref, y_ref, n128_ref):
        deg = hist_ref[...][:, 0]
        nrm = lax.rsqrt(jnp.maximum(deg, 1.0))
        n128_ref[...] = jnp.broadcast_to(nrm[:, None], (rb, CH))
        y_ref[...] = x_ref[...] * nrm[:, None]

    return pl.pallas_call(
        body,
        grid=(npad // rb,),
        in_specs=[
            pl.BlockSpec((rb, CH), lambda i: (i, 0)),
            pl.BlockSpec((rb, d), lambda i: (i, 0)),
        ],
        out_specs=[
            pl.BlockSpec((rb, d), lambda i: (i, 0)),
            pl.BlockSpec((rb, CH), lambda i: (i, 0)),
        ],
        out_shape=[
            jax.ShapeDtypeStruct((npad, d), jnp.float32),
            jax.ShapeDtypeStruct((npad, CH), jnp.float32),
        ],
    )


def _make_agg_kernel(npad, d, ems, bt):
    nec = ems // CH
    ngr = nec // EG
    ngt = bt // CH // NS
    npc = npad // NC
    rpt = npc // NS

    @functools.partial(
        pl.kernel,
        out_type=jax.ShapeDtypeStruct((NC, bt, d), jnp.float32),
        mesh=_sc_mesh(),
        scratch_types=[
            pltpu.VMEM((EG, CH), jnp.int32),
            pltpu.VMEM((EG, CH), jnp.int32),
            pltpu.VMEM((CH, d), jnp.float32),
            pltpu.VMEM((CH, d), jnp.float32),
            pltpu.VMEM((CH,), jnp.int32),
            pltpu.VMEM_SHARED((npad // NC + 8, d), jnp.float32),
            pltpu.SemaphoreType.DMA,
            pltpu.SemaphoreType.DMA,
        ],
    )
    def agg_kernel(y, esrc, edst, idxg, aggg_out,
                   sgrp, dgrp, buf, zbuf, gidx, aggS, sem0, sem1):
        c = lax.axis_index("c")
        s = lax.axis_index("s")
        lo = c * npc
        lanes = lax.broadcasted_iota(jnp.int32, (LN,), 0)
        zeros = jnp.zeros((LN,), jnp.float32)
        npl = d // LN

        def zb(i, _):
            zbuf[i // npl, pl.ds((i % npl) * LN, LN)] = zeros
            return 0

        lax.fori_loop(0, CH * npl, zb, 0)
        _zero_slice(zbuf, aggS, gidx, s, rpt, lanes)

        @pl.when(s == 0)
        def _():
            _fill_rows(gidx, npc - CH + 8, lanes)
            pltpu.sync_copy(zbuf, aggS.at[gidx])

        plsc.subcore_barrier()

        bufs = (buf, zbuf)
        sems = (sem0, sem1)

        def egrp(g, _):
            pltpu.sync_copy(esrc.at[s, pl.ds(g * EG, EG)], sgrp)
            pltpu.sync_copy(edst.at[s, pl.ds(g * EG, EG)], dgrp)
            _localize(dgrp, EG, lo, npc, lanes)

            cps = [None, None]
            cps[0] = pltpu.async_copy(y.at[sgrp.at[0]], bufs[0], sems[0])
            for j in range(EG):
                if j + 1 < EG:
                    nb = (j + 1) % 2
                    cps[nb] = pltpu.async_copy(
                        y.at[sgrp.at[j + 1]], bufs[nb], sems[nb]
                    )
                cps[j % 2].wait()
                pltpu.sync_copy(bufs[j % 2], aggS.at[dgrp.at[j]], add=True)
            return 0

        lax.fori_loop(0, ngr, egrp, 0)
        plsc.subcore_barrier()

        def gbody(k, _):
            j = s * ngt + k
            pltpu.sync_copy(idxg.at[j], gidx)

            def lxb(i, _):
                v = gidx[pl.ds(i * LN, LN)] - lo
                ok = (v >= 0) & (v < npc)
                gidx[pl.ds(i * LN, LN)] = jnp.where(ok, v, npc)
                return 0

            lax.fori_loop(0, CH // LN, lxb, 0)
            pltpu.async_copy(aggS.at[gidx], zbuf, sem1).wait()
            pltpu.sync_copy(zbuf, aggg_out.at[c, pl.ds(j * CH, CH)])
            return 0

        lax.fori_loop(0, ngt, gbody, 0)

    return agg_kernel


def _make_normg_kernel(npad, bt):
    ngt = bt // CH // NS
    half = ngt // NC

    @functools.partial(
        pl.kernel,
        out_type=jax.ShapeDtypeStruct((bt, CH), jnp.float32),
        mesh=_sc_mesh(),
        scratch_types=[
            pltpu.VMEM((CH,), jnp.int32),
            pltpu.VMEM((CH, CH), jnp.float32),
            pltpu.SemaphoreType.DMA,
        ],
    )
    def normg_kernel(n128, idxg, out_hbm, gidx, nbuf, sem):
        c = lax.axis_index("c")
        s = lax.axis_index("s")

        def gbody(k, _):
            j = s * ngt + c * half + k
            pltpu.sync_copy(idxg.at[j], gidx)
            pltpu.async_copy(n128.at[gidx], nbuf, sem).wait()
            pltpu.sync_copy(nbuf, out_hbm.at[pl.ds(j * CH, CH)])
            return 0

        lax.fori_loop(0, half, gbody, 0)

    return normg_kernel


def _make_mlp(bt, d, h1, npc):
    b = bt // 2

    def body(aggg_ref, normg_ref, idxg_ref, wg_ref, bg_ref, w1_ref, b1_ref,
             w2_ref, b2_ref, out_ref):
        nrm = normg_ref[...][:, :1]
        own0 = idxg_ref[...] < npc
        a = jnp.where(own0, aggg_ref[0], aggg_ref[1]) * nrm
        h = (
            jnp.dot(a, wg_ref[...], preferred_element_type=jnp.float32)
            + bg_ref[...]
        )
        h = jnp.maximum(h, 0.0)
        hf = h[:b]
        hs = h[b:]
        z = (
            jnp.dot(hf, w1_ref[:d, :], preferred_element_type=jnp.float32)
            + jnp.dot(hs, w1_ref[d:, :], preferred_element_type=jnp.float32)
            + b1_ref[...]
        )
        z = jnp.maximum(z, 0.0)
        out_ref[...] = (
            jnp.dot(z, w2_ref[...], preferred_element_type=jnp.float32)
            + b2_ref[...]
        )

    return pl.pallas_call(
        body,
        out_shape=jax.ShapeDtypeStruct((b, 1), jnp.float32),
    )


def kernel(first_index, second_index, edge_index, emb_user, emb_item,
           W_gcn, b_gcn, W1, b1, W2, b2):
    nu, d = emb_user.shape
    ni = emb_item.shape[0]
    n = nu + ni
    e = edge_index.shape[1]
    bsz = first_index.shape[0]
    h1 = W1.shape[1]

    npad = -(-(n + 1) // (NC * NS * CH)) * (NC * NS * CH)
    epad = -(-e // (NS * CH * EG // 2)) * (NS * CH * EG // 2)
    ems = 2 * epad // NS
    nept = 2 * epad // NS
    bt = 2 * bsz

    idt = jnp.int32
    pad_e = jnp.full((epad - e,), n, dtype=idt)
    e0 = jnp.concatenate([edge_index[0].astype(idt), pad_e])
    e1 = jnp.concatenate([edge_index[1].astype(idt), pad_e])
    edset = jnp.concatenate([e0, e1]).reshape(NS * nept // CH, CH)
    esrc = jnp.concatenate(
        [e0.reshape(NS, -1), e1.reshape(NS, -1)], axis=1
    ).reshape(NS, ems // CH, CH)
    edst = jnp.concatenate(
        [e1.reshape(NS, -1), e0.reshape(NS, -1)], axis=1
    ).reshape(NS, ems // CH, CH)
    x = jnp.concatenate(
        [emb_user, emb_item, jnp.zeros((npad - n, d), jnp.float32)]
    )
    idxg_flat = jnp.concatenate(
        [first_index.astype(idt), second_index.astype(idt) + nu]
    )

    hist = _make_deg_kernel(npad, nept)(edset)
    y, n128 = _make_norm_scale(npad, d, 2048)(hist, x)
    aggg = _make_agg_kernel(npad, d, ems, bt)(
        y, esrc, edst, idxg_flat.reshape(bt // CH, CH)
    )
    normg = _make_normg_kernel(npad, bt)(n128, idxg_flat.reshape(bt // CH, CH))
    out = _make_mlp(bt, d, h1, npad // NC)(
        aggg,
        normg,
        idxg_flat.reshape(bt, 1),
        W_gcn,
        b_gcn.reshape(1, d),
        W1,
        b1.reshape(1, h1),
        W2,
        b2.reshape(1, 1),
    )
    return out.reshape(bsz)


# speedup vs baseline: 3.4250x; 1.0041x over previous
import functools

import jax
import jax.numpy as jnp
from jax import lax
from jax.experimental import pallas as pl
from jax.experimental.pallas import tpu as pltpu
from jax.experimental.pallas import tpu_sc as plsc

NC = 2
NS = 16
LN = 16
CH = 128
EG = 32


def _sc_mesh():
    return plsc.VectorSubcoreMesh(
        core_axis_name="c", subcore_axis_name="s", num_cores=NC, num_subcores=NS
    )


def _fill_rows(idxb, base, lanes):
    def fi(i, _):
        idxb[pl.ds(i * LN, LN)] = base + i * LN + lanes
        return 0

    lax.fori_loop(0, CH // LN, fi, 0)


def _zero_slice(zbuf, shared, gidx, s, rpt, lanes):
    def zc(t, _):
        _fill_rows(gidx, s * rpt + t * CH, lanes)
        pltpu.sync_copy(zbuf, shared.at[gidx])
        return 0

    lax.fori_loop(0, rpt // CH, zc, 0)


def _localize(buf2d, rows, lo, npc, lanes_unused):
    vec = CH // LN

    def txb(i, _):
        j = i // vec
        k = (i % vec) * LN
        v = buf2d[j, pl.ds(k, LN)] - lo
        ok = (v >= 0) & (v < npc)
        buf2d[j, pl.ds(k, LN)] = jnp.where(ok, v, npc)
        return 0

    lax.fori_loop(0, rows * vec, txb, 0)


def _make_deg_kernel(npad, nept):
    nec = nept // CH
    ngr = nec // EG
    npc = npad // NC
    rpt = npc // NS

    @functools.partial(
        pl.kernel,
        out_type=jax.ShapeDtypeStruct((npad, CH), jnp.float32),
        mesh=_sc_mesh(),
        scratch_types=[
            pltpu.VMEM((EG, CH), jnp.int32),
            pltpu.VMEM((CH, CH), jnp.float32),
            pltpu.VMEM((CH, CH), jnp.float32),
            pltpu.VMEM((CH,), jnp.int32),
            pltpu.VMEM_SHARED((npad // NC + 8, CH), jnp.float32),
        ],
    )
    def deg_kernel(edset, out_hbm, sgrp, ones_b, zbuf, gidx, histS):
        c = lax.axis_index("c")
        s = lax.axis_index("s")
        lo = c * npc
        lanes = lax.broadcasted_iota(jnp.int32, (LN,), 0)
        zeros = jnp.zeros((LN,), jnp.float32)
        npl = CH // LN

        def fb(i, _):
            zbuf[i // npl, pl.ds((i % npl) * LN, LN)] = zeros
            ones_b[i // npl, pl.ds((i % npl) * LN, LN)] = zeros + 1.0
            return 0

        lax.fori_loop(0, CH * npl, fb, 0)
        _zero_slice(zbuf, histS, gidx, s, rpt, lanes)

        @pl.when(s == 0)
        def _():
            _fill_rows(gidx, npc - CH + 8, lanes)
            pltpu.sync_copy(zbuf, histS.at[gidx])

        plsc.subcore_barrier()

        def egrp(g, _):
            pltpu.sync_copy(edset.at[pl.ds((s * ngr + g) * EG, EG)], sgrp)
            _localize(sgrp, EG, lo, npc, lanes)

            def body(j, _):
                pltpu.sync_copy(ones_b, histS.at[sgrp.at[j]], add=True)
                return 0

            lax.fori_loop(0, EG, body, 0)
            return 0

        lax.fori_loop(0, ngr, egrp, 0)
        plsc.subcore_barrier()

        def co(t, _):
            r = s * rpt + t * CH
            _fill_rows(gidx, r, lanes)
            pltpu.sync_copy(histS.at[gidx], zbuf)
            pltpu.sync_copy(zbuf, out_hbm.at[pl.ds(c * npc + r, CH)])
            return 0

        lax.fori_loop(0, rpt // CH, co, 0)

    return deg_kernel


def _make_norm_scale(npad, d, rb):
    def body(hist_ref, x_ref, y_ref, n128_ref):
        deg = hist_ref[...][:, 0]
        nrm = lax.rsqrt(jnp.maximum(deg, 1.0))
        n128_ref[...] = jnp.broadcast_to(nrm[:, None], (rb, CH))
        y_ref[...] = x_ref[...] * nrm[:, None]

    return pl.pallas_call(
        body,
        grid=(npad // rb,),
        in_specs=[
            pl.BlockSpec((rb, CH), lambda i: (i, 0)),
            pl.BlockSpec((rb, d), lambda i: (i, 0)),
        ],
        out_specs=[
            pl.BlockSpec((rb, d), lambda i: (i, 0)),
            pl.BlockSpec((rb, CH), lambda i: (i, 0)),
        ],
        out_shape=[
            jax.ShapeDtypeStruct((npad, d), jnp.float32),
            jax.ShapeDtypeStruct((npad, CH), jnp.float32),
        ],
    )


def _make_agg_kernel(npad, d, ems, bt):
    nec = ems // CH
    ngr = nec // EG
    ngt = bt // CH // NS
    npc = npad // NC
    rpt = npc // NS

    @functools.partial(
        pl.kernel,
        out_type=jax.ShapeDtypeStruct((NC, bt, d), jnp.float32),
        mesh=_sc_mesh(),
        scratch_types=[
            pltpu.VMEM((EG, CH), jnp.int32),
            pltpu.VMEM((EG, CH), jnp.int32),
            pltpu.VMEM((CH, d), jnp.float32),
            pltpu.VMEM((CH, d), jnp.float32),
            pltpu.VMEM((CH,), jnp.int32),
            pltpu.VMEM_SHARED((npad // NC + 8, d), jnp.float32),
            pltpu.SemaphoreType.DMA,
            pltpu.SemaphoreType.DMA,
        ],
    )
    def agg_kernel(y, esrc, edst, idxg, aggg_out,
                   sgrp, dgrp, buf, zbuf, gidx, aggS, sem0, sem1):
        c = lax.axis_index("c")
        s = lax.axis_index("s")
        lo = c * npc
        lanes = lax.broadcasted_iota(jnp.int32, (LN,), 0)
        zeros = jnp.zeros((LN,), jnp.float32)
        npl = d // LN

        def zb(i, _):
            zbuf[i // npl, pl.ds((i % npl) * LN, LN)] = zeros
            return 0

        lax.fori_loop(0, CH * npl, zb, 0)
        _zero_slice(zbuf, aggS, gidx, s, rpt, lanes)

        @pl.when(s == 0)
        def _():
            _fill_rows(gidx, npc - CH + 8, lanes)
            pltpu.sync_copy(zbuf, aggS.at[gidx])

        plsc.subcore_barrier()

        bufs = (buf, zbuf)
        sems = (sem0, sem1)

        def egrp(g, _):
            pltpu.sync_copy(esrc.at[s, pl.ds(g * EG, EG)], sgrp)
            pltpu.sync_copy(edst.at[s, pl.ds(g * EG, EG)], dgrp)
            _localize(dgrp, EG, lo, npc, lanes)

            cps = [None, None]
            cps[0] = pltpu.async_copy(y.at[sgrp.at[0]], bufs[0], sems[0])
            for j in range(EG):
                if j + 1 < EG:
                    nb = (j + 1) % 2
                    cps[nb] = pltpu.async_copy(
                        y.at[sgrp.at[j + 1]], bufs[nb], sems[nb]
                    )
                cps[j % 2].wait()
                pltpu.sync_copy(bufs[j % 2], aggS.at[dgrp.at[j]], add=True)
            return 0

        lax.fori_loop(0, ngr, egrp, 0)
        plsc.subcore_barrier()

        def gbody(k, _):
            j = s * ngt + k
            pltpu.sync_copy(idxg.at[j], gidx)

            def lxb(i, _):
                v = gidx[pl.ds(i * LN, LN)] - lo
                ok = (v >= 0) & (v < npc)
                gidx[pl.ds(i * LN, LN)] = jnp.where(ok, v, npc)
                return 0

            lax.fori_loop(0, CH // LN, lxb, 0)
            pltpu.async_copy(aggS.at[gidx], zbuf, sem1).wait()
            pltpu.sync_copy(zbuf, aggg_out.at[c, pl.ds(j * CH, CH)])
            return 0

        lax.fori_loop(0, ngt, gbody, 0)

    return agg_kernel


def _make_normg_kernel(npad, bt):
    ngt = bt // CH // NS
    half = ngt // NC

    @functools.partial(
        pl.kernel,
        out_type=jax.ShapeDtypeStruct((bt, CH), jnp.float32),
        mesh=_sc_mesh(),
        scratch_types=[
            pltpu.VMEM((CH,), jnp.int32),
            pltpu.VMEM((CH, CH), jnp.float32),
            pltpu.SemaphoreType.DMA,
        ],
    )
    def normg_kernel(n128, idxg, out_hbm, gidx, nbuf, sem):
        c = lax.axis_index("c")
        s = lax.axis_index("s")

        def gbody(k, _):
            j = s * ngt + c * half + k
            pltpu.sync_copy(idxg.at[j], gidx)
            pltpu.async_copy(n128.at[gidx], nbuf, sem).wait()
            pltpu.sync_copy(nbuf, out_hbm.at[pl.ds(j * CH, CH)])
            return 0

        lax.fori_loop(0, half, gbody, 0)

    return normg_kernel


def _make_mlp(bt, d, h1, npc):
    b = bt // 2

    def body(aggg_ref, normg_ref, idxg_ref, wg_ref, bg_ref, w1_ref, b1_ref,
             w2_ref, b2_ref, out_ref):
        nrm = normg_ref[...][:, :1]
        own0 = idxg_ref[...] < npc
        a = jnp.where(own0, aggg_ref[0], aggg_ref[1]) * nrm
        h = (
            jnp.dot(a, wg_ref[...], preferred_element_type=jnp.float32)
            + bg_ref[...]
        )
        h = jnp.maximum(h, 0.0)
        hf = h[:b]
        hs = h[b:]
        z = (
            jnp.dot(hf, w1_ref[:d, :], preferred_element_type=jnp.float32)
            + jnp.dot(hs, w1_ref[d:, :], preferred_element_type=jnp.float32)
            + b1_ref[...]
        )
        z = jnp.maximum(z, 0.0)
        out_ref[...] = (
            jnp.dot(z, w2_ref[...], preferred_element_type=jnp.float32)
            + b2_ref[...]
        )

    return pl.pallas_call(
        body,
        out_shape=jax.ShapeDtypeStruct((b, 1), jnp.float32),
    )


def kernel(first_index, second_index, edge_index, emb_user, emb_item,
           W_gcn, b_gcn, W1, b1, W2, b2):
    nu, d = emb_user.shape
    ni = emb_item.shape[0]
    n = nu + ni
    e = edge_index.shape[1]
    bsz = first_index.shape[0]
    h1 = W1.shape[1]

    npad = -(-(n + 1) // (NC * NS * CH)) * (NC * NS * CH)
    epad = -(-e // (NS * CH * EG // 2)) * (NS * CH * EG // 2)
    ems = 2 * epad // NS
    nept = 2 * epad // NS
    bt = 2 * bsz

    idt = jnp.int32
    pad_e = jnp.full((epad - e,), n, dtype=idt)
    e0 = jnp.concatenate([edge_index[0].astype(idt), pad_e])
    e1 = jnp.concatenate([edge_index[1].astype(idt), pad_e])
    edset = jnp.concatenate([e0, e1]).reshape(NS * nept // CH, CH)
    esrc = jnp.concatenate(
        [e0.reshape(NS, -1), e1.reshape(NS, -1)], axis=1
    ).reshape(NS, ems // CH, CH)
    edst = jnp.concatenate(
        [e1.reshape(NS, -1), e0.reshape(NS, -1)], axis=1
    ).reshape(NS, ems // CH, CH)
    x = jnp.concatenate(
        [emb_user, emb_item, jnp.zeros((npad - n, d), jnp.float32)]
    )
    idxg_flat = jnp.concatenate(
        [first_index.astype(idt), second_index.astype(idt) + nu]
    )

    hist = _make_deg_kernel(npad, nept)(edset)
    y, n128 = _make_norm_scale(npad, d, 2048)(hist, x)
    aggg = _make_agg_kernel(npad, d, ems, bt)(
        y, esrc, edst, idxg_flat.reshape(bt // CH, CH)
    )
    normg = _make_normg_kernel(npad, bt)(n128, idxg_flat.reshape(bt // CH, CH))
    out = _make_mlp(bt, d, h1, npad // NC)(
        aggg,
        normg,
        idxg_flat.reshape(bt, 1),
        W_gcn,
        b_gcn.reshape(1, d),
        W1,
        b1.reshape(1, h1),
        W2,
        b2.reshape(1, 1),
    )
    return out.reshape(bsz)
